# baseline (speedup 1.0000x reference)
import functools

import jax
import jax.numpy as jnp
from jax import lax
from jax.experimental import pallas as pl
from jax.experimental.pallas import tpu as pltpu
from jax.experimental.pallas import tpu_sc as plsc

_ALPHA = 0.2
_NC = 2
_NS = 16
_LANES = 16


_GCHUNK = 128


def _sc_body(B, S, C, rpw, neighbors_flat, indexes, targets, logits_flat,
             nb_out, tlog_out, idx_v, jidx_v, flat_v, nb_v, t_v, tl_v, sem):
    wid = lax.axis_index("s") * _NC + lax.axis_index("c")
    base = wid * rpw
    pltpu.sync_copy(indexes.at[pl.ds(base, rpw)], idx_v)
    pltpu.sync_copy(indexes, jidx_v)
    pltpu.sync_copy(targets.at[pl.ds(base, rpw)], t_v)
    idx_chunk = idx_v[...]
    dn = lax.GatherDimensionNumbers(
        offset_dims=(), collapsed_slice_dims=(0,), start_index_map=(0,))
    for b in range(rpw):
        rowbase = lax.gather(
            idx_chunk, jnp.full((_LANES, 1), b, jnp.int32), dn,
            slice_sizes=(1,),
            mode=lax.GatherScatterMode.PROMISE_IN_BOUNDS) * S

        def body(jj, carry, b=b, rowbase=rowbase):
            flat_v[pl.ds(b * B + jj * _LANES, _LANES)] = (
                rowbase + jidx_v[pl.ds(jj * _LANES, _LANES)])
            return carry

        lax.fori_loop(0, B // _LANES, body, 0)
    copies = []
    for k in range(rpw * B // _GCHUNK):
        copies.append(pltpu.async_copy(
            neighbors_flat.at[flat_v.at[pl.ds(k * _GCHUNK, _GCHUNK)]],
            nb_v.at[pl.ds(k * _GCHUNK, _GCHUNK)], sem))
    for cp in copies:
        cp.wait()
    pltpu.sync_copy(nb_v, nb_out.at[pl.ds(base * B, rpw * B)])
    bvec = base + lax.iota(jnp.int32, _LANES)
    flat = bvec * C + t_v[...]
    pltpu.async_copy(logits_flat.at[flat], tl_v, sem).wait()
    pltpu.sync_copy(tl_v, tlog_out.at[pl.ds(base, rpw)])


@functools.lru_cache(maxsize=None)
def _make_sc(B, S, C):
    rpw = B // (_NC * _NS)
    mesh = plsc.VectorSubcoreMesh(core_axis_name="c", subcore_axis_name="s")
    return pl.kernel(
        functools.partial(_sc_body, B, S, C, rpw),
        mesh=mesh,
        out_type=[
            jax.ShapeDtypeStruct((B * B,), jnp.int32),
            jax.ShapeDtypeStruct((B,), jnp.float32),
        ],
        scratch_types=[
            pltpu.VMEM((rpw,), jnp.int32),
            pltpu.VMEM((B,), jnp.int32),
            pltpu.VMEM((rpw * B,), jnp.int32),
            pltpu.VMEM((rpw * B,), jnp.int32),
            pltpu.VMEM((rpw,), jnp.int32),
            pltpu.VMEM((rpw,), jnp.float32),
            pltpu.SemaphoreType.DMA,
        ],
    )


def _tc_body(x_ref, idxc_ref, idxr_ref, nb_ref, tlog_ref, out_ref):
    x = x_ref[...]
    m = jnp.max(x, axis=1, keepdims=True)
    e = jnp.exp(x - m)
    se = jnp.sum(e, axis=1, keepdims=True)
    lse = m + jnp.log(se)
    p = e / se
    lp = x - lse
    g = lax.dot_general(lp, p, (((1,), (1,)), ((), ())),
                        preferred_element_type=jnp.float32)
    b = x.shape[0]
    eq = idxc_ref[...] == idxr_ref[...]
    ii = lax.broadcasted_iota(jnp.int32, (b, b), 0)
    jj = lax.broadcasted_iota(jnp.int32, (b, b), 1)
    dup = jnp.max(jnp.where(eq & (ii < jj), 1.0, 0.0), axis=0, keepdims=True)
    nbf = jnp.where(nb_ref[...] != 0, 1.0, 0.0)
    mmat = nbf * (1.0 - dup)
    cnt = jnp.maximum(jnp.sum(mmat, axis=1, keepdims=True), 1.0)
    neigh = jnp.sum(mmat * g, axis=1, keepdims=True) / cnt
    lpt = tlog_ref[...] - lse
    loss = jnp.mean(-_ALPHA * lpt - (1.0 - _ALPHA) * neigh)
    out_ref[...] = jnp.reshape(loss, (1, 1))


def _tc_call(x, idxc, idxr, nb, tlog, interpret=False):
    return pl.pallas_call(
        _tc_body,
        out_shape=jax.ShapeDtypeStruct((1, 1), jnp.float32),
        interpret=interpret,
    )(x, idxc, idxr, nb, tlog)


def kernel(inputs, inputs_logits, targets, indexes, neighbors, neighbor_dists):
    del inputs, neighbor_dists
    B, C = inputs_logits.shape
    S = neighbors.shape[1]
    nb, tlog = _make_sc(B, S, C)(
        neighbors.reshape(-1), indexes, targets, inputs_logits.reshape(-1))
    loss = _tc_call(inputs_logits, indexes.reshape(B, 1), indexes.reshape(1, B),
                    nb.reshape(B, B), tlog.reshape(B, 1))
    return loss[0, 0]


# speedup vs baseline: 2.0182x; 2.0182x over previous
import functools

import jax
import jax.numpy as jnp
from jax import lax
from jax.experimental import pallas as pl
from jax.experimental.pallas import tpu as pltpu
from jax.experimental.pallas import tpu_sc as plsc

_ALPHA = 0.2
_NC = 2
_NS = 16
_LANES = 16
_GCHUNK = 128


def _sc_body(B, S, rpw, neighbors, indexes, nb_out, rows_lin,
             idx_v, jidx_v, rows_v, flat_v, nb_v, sem):
    wid = lax.axis_index("s") * _NC + lax.axis_index("c")
    base = wid * rpw
    pltpu.sync_copy(indexes.at[pl.ds(base, rpw)], idx_v)
    pltpu.sync_copy(indexes, jidx_v)
    pltpu.async_copy(neighbors.at[idx_v], rows_v, sem).wait()
    for b in range(rpw):
        pltpu.sync_copy(rows_v.at[b], rows_lin.at[pl.ds((base + b) * S, S)])
    for b in range(rpw):
        rowoff = (base + b) * S

        def body(jj, carry, rowoff=rowoff, b=b):
            flat_v[pl.ds(b * B + jj * _LANES, _LANES)] = (
                rowoff + jidx_v[pl.ds(jj * _LANES, _LANES)])
            return carry

        lax.fori_loop(0, B // _LANES, body, 0)
    copies = []
    for k in range(rpw * B // _GCHUNK):
        copies.append(pltpu.async_copy(
            rows_lin.at[flat_v.at[pl.ds(k * _GCHUNK, _GCHUNK)]],
            nb_v.at[pl.ds(k * _GCHUNK, _GCHUNK)], sem))
    for cp in copies:
        cp.wait()
    pltpu.sync_copy(nb_v, nb_out.at[pl.ds(base * B, rpw * B)])


@functools.lru_cache(maxsize=None)
def _make_sc(B, S):
    rpw = B // (_NC * _NS)
    mesh = plsc.VectorSubcoreMesh(core_axis_name="c", subcore_axis_name="s")
    return pl.kernel(
        functools.partial(_sc_body, B, S, rpw),
        mesh=mesh,
        out_type=[
            jax.ShapeDtypeStruct((B * B,), jnp.int32),
            jax.ShapeDtypeStruct((B * S,), jnp.int32),
        ],
        scratch_types=[
            pltpu.VMEM((rpw,), jnp.int32),
            pltpu.VMEM((B,), jnp.int32),
            pltpu.VMEM((rpw, S), jnp.int32),
            pltpu.VMEM((rpw * B,), jnp.int32),
            pltpu.VMEM((rpw * B,), jnp.int32),
            pltpu.SemaphoreType.DMA,
        ],
    )


def _tc_body(x_ref, idxc_ref, idxr_ref, tgt_ref, nb_ref, out_ref):
    x = x_ref[...]
    m = jnp.max(x, axis=1, keepdims=True)
    e = jnp.exp(x - m)
    se = jnp.sum(e, axis=1, keepdims=True)
    lse = m + jnp.log(se)
    p = e / se
    lp = x - lse
    g = lax.dot_general(lp, p, (((1,), (1,)), ((), ())),
                        preferred_element_type=jnp.float32)
    b, c = x.shape
    eq = idxc_ref[...] == idxr_ref[...]
    ii = lax.broadcasted_iota(jnp.int32, (b, b), 0)
    jj = lax.broadcasted_iota(jnp.int32, (b, b), 1)
    dup = jnp.max(jnp.where(eq & (ii < jj), 1.0, 0.0), axis=0, keepdims=True)
    nbf = jnp.where(nb_ref[...] != 0, 1.0, 0.0)
    mmat = nbf * (1.0 - dup)
    cnt = jnp.maximum(jnp.sum(mmat, axis=1, keepdims=True), 1.0)
    neigh = jnp.sum(mmat * g, axis=1, keepdims=True) / cnt
    ci = lax.broadcasted_iota(jnp.int32, (b, c), 1)
    tx = jnp.sum(jnp.where(ci == tgt_ref[...], x, 0.0), axis=1, keepdims=True)
    lpt = tx - lse
    loss = jnp.mean(-_ALPHA * lpt - (1.0 - _ALPHA) * neigh)
    out_ref[...] = jnp.reshape(loss, (1, 1))


def _tc_call(x, idxc, idxr, tgt, nb, interpret=False):
    return pl.pallas_call(
        _tc_body,
        out_shape=jax.ShapeDtypeStruct((1, 1), jnp.float32),
        interpret=interpret,
    )(x, idxc, idxr, tgt, nb)


def kernel(inputs, inputs_logits, targets, indexes, neighbors, neighbor_dists):
    del inputs, neighbor_dists
    B, _ = inputs_logits.shape
    S = neighbors.shape[1]
    nb, _ = _make_sc(B, S)(neighbors, indexes)
    loss = _tc_call(inputs_logits, indexes.reshape(B, 1), indexes.reshape(1, B),
                    targets.reshape(B, 1), nb.reshape(B, B))
    return loss[0, 0]


# speedup vs baseline: 2.5906x; 1.2836x over previous
import functools

import jax
import jax.numpy as jnp
from jax import lax
from jax.experimental import pallas as pl
from jax.experimental.pallas import tpu as pltpu
from jax.experimental.pallas import tpu_sc as plsc

_ALPHA = 0.2
_NC = 2
_NS = 16
_LANES = 16
_GCHUNK = 128


def _sc_body(B, S, rpw, neighbors, indexes, nb_out,
             idx_v, jidx_v, rows_v, flat_v, nb_v, shared,
             semr0, semr1, sems0, sems1, semg, semo):
    sidx = lax.axis_index("s")
    wid = sidx * _NC + lax.axis_index("c")
    base = wid * rpw
    half = rpw // 2
    qr = rpw // 4
    sems = [sems0, sems1]
    pltpu.sync_copy(indexes.at[pl.ds(base, rpw)], idx_v)
    pltpu.sync_copy(indexes, jidx_v)
    rcp = [pltpu.async_copy(neighbors.at[idx_v.at[pl.ds(h * half, half)]],
                            rows_v.at[pl.ds(h * half, half)], sem)
           for h, sem in ((0, semr0), (1, semr1))]
    rowoffs = [(sidx * 2 * qr + ((b // qr) % 2) * qr + b % qr) * S
               for b in range(rpw)]

    def body(jj, carry):
        off = jj * _LANES
        chunk = jidx_v[pl.ds(off, _LANES)]
        for b in range(rpw):
            flat_v[pl.ds(b * B + off, _LANES)] = chunk + rowoffs[b]
        return carry

    lax.fori_loop(0, B // _LANES, body, 0)

    def fire_stage(p):
        reg = p % 2
        return [pltpu.async_copy(
            rows_v.at[p * qr + bl],
            shared.at[pl.ds((sidx * 2 * qr + reg * qr + bl) * S, S)],
            sems[reg])
            for bl in range(qr)]

    def fire_gathers(p):
        out = []
        for k in range(qr * B // _GCHUNK):
            off = p * qr * B + k * _GCHUNK
            out.append(pltpu.async_copy(
                shared.at[flat_v.at[pl.ds(off, _GCHUNK)]],
                nb_v.at[pl.ds(off, _GCHUNK)], semg))
        return out

    def fire_out(p):
        return pltpu.async_copy(
            nb_v.at[pl.ds(p * qr * B, qr * B)],
            nb_out.at[pl.ds(base * B + p * qr * B, qr * B)], semo)

    def drain(cps):
        for cp in cps:
            cp.wait()

    outs = []
    rcp[0].wait()
    s0 = fire_stage(0)
    s1 = fire_stage(1)
    drain(s0)
    g0 = fire_gathers(0)
    rcp[1].wait()
    drain(g0)
    outs.append(fire_out(0))
    s2 = fire_stage(2)
    drain(s1)
    g1 = fire_gathers(1)
    drain(g1)
    outs.append(fire_out(1))
    s3 = fire_stage(3)
    drain(s2)
    g2 = fire_gathers(2)
    drain(g2)
    outs.append(fire_out(2))
    drain(s3)
    g3 = fire_gathers(3)
    drain(g3)
    outs.append(fire_out(3))
    drain(outs)


@functools.lru_cache(maxsize=None)
def _make_sc(B, S):
    rpw = B // (_NC * _NS)
    mesh = plsc.VectorSubcoreMesh(core_axis_name="c", subcore_axis_name="s")
    return pl.kernel(
        functools.partial(_sc_body, B, S, rpw),
        mesh=mesh,
        out_type=[
            jax.ShapeDtypeStruct((B * B,), jnp.int32),
        ],
        scratch_types=[
            pltpu.VMEM((rpw,), jnp.int32),
            pltpu.VMEM((B,), jnp.int32),
            pltpu.VMEM((rpw, S), jnp.int32),
            pltpu.VMEM((rpw * B,), jnp.int32),
            pltpu.VMEM((rpw * B,), jnp.int32),
            pltpu.VMEM_SHARED((_NS * (rpw // 2) * S,), jnp.int32),
            pltpu.SemaphoreType.DMA,
            pltpu.SemaphoreType.DMA,
            pltpu.SemaphoreType.DMA,
            pltpu.SemaphoreType.DMA,
            pltpu.SemaphoreType.DMA,
            pltpu.SemaphoreType.DMA,
        ],
    )


def _tc_dense_body(x_ref, tgt_ref, g_ref, lpt_ref):
    x = x_ref[...]
    b, c = x.shape
    m = jnp.max(x, axis=1, keepdims=True)
    e = jnp.exp(x - m)
    se = jnp.sum(e, axis=1, keepdims=True)
    lse = m + jnp.log(se)
    p = e / se
    lp = x - lse
    g_ref[...] = lax.dot_general(
        lp.astype(jnp.bfloat16), p.astype(jnp.bfloat16),
        (((1,), (1,)), ((), ())),
        preferred_element_type=jnp.float32)
    ci = lax.broadcasted_iota(jnp.int32, (b, c), 1)
    tx = jnp.sum(jnp.where(ci == tgt_ref[...], x, 0.0), axis=1, keepdims=True)
    lpt_ref[...] = tx - lse


def _tc_dense(x, tgt, interpret=False):
    b = x.shape[0]
    return pl.pallas_call(
        _tc_dense_body,
        out_shape=[jax.ShapeDtypeStruct((b, b), jnp.float32),
                   jax.ShapeDtypeStruct((b, 1), jnp.float32)],
        interpret=interpret,
    )(x, tgt)


def _tc_combine_body(g_ref, lpt_ref, idxr_ref, nb_ref, out_ref):
    b = g_ref.shape[0]
    idxr = idxr_ref[...]
    eq = jnp.reshape(idxr, (b, 1)) == idxr
    ii = lax.broadcasted_iota(jnp.int32, (b, b), 0)
    jj = lax.broadcasted_iota(jnp.int32, (b, b), 1)
    dup = jnp.max(jnp.where(eq & (ii < jj), 1.0, 0.0), axis=0, keepdims=True)
    nb = jnp.reshape(nb_ref[...], (b, b))
    nbf = jnp.where(nb != 0, 1.0, 0.0)
    mmat = nbf * (1.0 - dup)
    cnt = jnp.maximum(jnp.sum(mmat, axis=1, keepdims=True), 1.0)
    neigh = jnp.sum(mmat * g_ref[...], axis=1, keepdims=True) / cnt
    loss = jnp.mean(-_ALPHA * lpt_ref[...] - (1.0 - _ALPHA) * neigh)
    out_ref[...] = jnp.reshape(loss, (1, 1))


def _tc_combine(g, lpt, idxr, nb, interpret=False):
    return pl.pallas_call(
        _tc_combine_body,
        out_shape=jax.ShapeDtypeStruct((1, 1), jnp.float32),
        interpret=interpret,
    )(g, lpt, idxr, nb)


def kernel(inputs, inputs_logits, targets, indexes, neighbors, neighbor_dists):
    del inputs, neighbor_dists
    B, _ = inputs_logits.shape
    S = neighbors.shape[1]
    (nb,) = _make_sc(B, S)(neighbors, indexes)
    g, lpt = _tc_dense(inputs_logits, targets.reshape(B, 1))
    loss = _tc_combine(g, lpt, indexes.reshape(1, B), nb)
    return loss[0, 0]


# speedup vs baseline: 2.6533x; 1.0242x over previous
import functools

import jax
import jax.numpy as jnp
from jax import lax
from jax.experimental import pallas as pl
from jax.experimental.pallas import tpu as pltpu
from jax.experimental.pallas import tpu_sc as plsc

_ALPHA = 0.2
_NC = 2
_NS = 16
_LANES = 16
_GCHUNK = 2048


def _sc_body(B, S, rpw, neighbors, indexes, nb_out,
             idx_v, jidx_v, rows_v, flat_v, nb_v, shared,
             semr0, semr1, sems0, sems1, semg, semo):
    sidx = lax.axis_index("s")
    wid = sidx * _NC + lax.axis_index("c")
    base = wid * rpw
    half = rpw // 2
    qr = rpw // 4
    sems = [sems0, sems1]
    pltpu.sync_copy(indexes.at[pl.ds(base, rpw)], idx_v)
    pltpu.sync_copy(indexes, jidx_v)
    rcp = [pltpu.async_copy(neighbors.at[idx_v.at[pl.ds(h * half, half)]],
                            rows_v.at[pl.ds(h * half, half)], sem)
           for h, sem in ((0, semr0), (1, semr1))]
    rowoffs = [(sidx * 2 * qr + ((b // qr) % 2) * qr + b % qr) * S
               for b in range(rpw)]

    def body(jj, carry):
        off = jj * _LANES
        chunk = jidx_v[pl.ds(off, _LANES)]
        for b in range(rpw):
            flat_v[pl.ds(b * B + off, _LANES)] = chunk + rowoffs[b]
        return carry

    lax.fori_loop(0, B // _LANES, body, 0)

    def fire_stage(p):
        reg = p % 2
        return [pltpu.async_copy(
            rows_v.at[p * qr + bl],
            shared.at[pl.ds((sidx * 2 * qr + reg * qr + bl) * S, S)],
            sems[reg])
            for bl in range(qr)]

    def fire_gathers(p):
        out = []
        for k in range(qr * B // _GCHUNK):
            off = p * qr * B + k * _GCHUNK
            out.append(pltpu.async_copy(
                shared.at[flat_v.at[pl.ds(off, _GCHUNK)]],
                nb_v.at[pl.ds(off, _GCHUNK)], semg))
        return out

    def fire_out(p):
        return pltpu.async_copy(
            nb_v.at[pl.ds(p * qr * B, qr * B)],
            nb_out.at[pl.ds(base * B + p * qr * B, qr * B)], semo)

    def drain(cps):
        for cp in cps:
            cp.wait()

    outs = []
    rcp[0].wait()
    s0 = fire_stage(0)
    s1 = fire_stage(1)
    drain(s0)
    g0 = fire_gathers(0)
    rcp[1].wait()
    drain(g0)
    outs.append(fire_out(0))
    s2 = fire_stage(2)
    drain(s1)
    g1 = fire_gathers(1)
    drain(g1)
    outs.append(fire_out(1))
    s3 = fire_stage(3)
    drain(s2)
    g2 = fire_gathers(2)
    drain(g2)
    outs.append(fire_out(2))
    drain(s3)
    g3 = fire_gathers(3)
    drain(g3)
    outs.append(fire_out(3))
    drain(outs)


@functools.lru_cache(maxsize=None)
def _make_sc(B, S):
    rpw = B // (_NC * _NS)
    mesh = plsc.VectorSubcoreMesh(core_axis_name="c", subcore_axis_name="s")
    return pl.kernel(
        functools.partial(_sc_body, B, S, rpw),
        mesh=mesh,
        out_type=[
            jax.ShapeDtypeStruct((B * B,), jnp.int32),
        ],
        scratch_types=[
            pltpu.VMEM((rpw,), jnp.int32),
            pltpu.VMEM((B,), jnp.int32),
            pltpu.VMEM((rpw, S), jnp.int32),
            pltpu.VMEM((rpw * B,), jnp.int32),
            pltpu.VMEM((rpw * B,), jnp.int32),
            pltpu.VMEM_SHARED((_NS * (rpw // 2) * S,), jnp.int32),
            pltpu.SemaphoreType.DMA,
            pltpu.SemaphoreType.DMA,
            pltpu.SemaphoreType.DMA,
            pltpu.SemaphoreType.DMA,
            pltpu.SemaphoreType.DMA,
            pltpu.SemaphoreType.DMA,
        ],
    )


def _tc_dense_body(x_ref, tgt_ref, g_ref, lpt_ref):
    x = x_ref[...]
    b, c = x.shape
    m = jnp.max(x, axis=1, keepdims=True)
    e = jnp.exp(x - m)
    se = jnp.sum(e, axis=1, keepdims=True)
    lse = m + jnp.log(se)
    p = e / se
    lp = x - lse
    g_ref[...] = lax.dot_general(
        lp.astype(jnp.bfloat16), p.astype(jnp.bfloat16),
        (((1,), (1,)), ((), ())),
        preferred_element_type=jnp.float32)
    ci = lax.broadcasted_iota(jnp.int32, (b, c), 1)
    tx = jnp.sum(jnp.where(ci == tgt_ref[...], x, 0.0), axis=1, keepdims=True)
    lpt_ref[...] = tx - lse


def _tc_dense(x, tgt, interpret=False):
    b = x.shape[0]
    return pl.pallas_call(
        _tc_dense_body,
        out_shape=[jax.ShapeDtypeStruct((b, b), jnp.float32),
                   jax.ShapeDtypeStruct((b, 1), jnp.float32)],
        interpret=interpret,
    )(x, tgt)


def _tc_combine_body(g_ref, lpt_ref, idxr_ref, nb_ref, out_ref):
    b = g_ref.shape[0]
    idxr = idxr_ref[...]
    eq = jnp.reshape(idxr, (b, 1)) == idxr
    ii = lax.broadcasted_iota(jnp.int32, (b, b), 0)
    jj = lax.broadcasted_iota(jnp.int32, (b, b), 1)
    dup = jnp.max(jnp.where(eq & (ii < jj), 1.0, 0.0), axis=0, keepdims=True)
    nb = jnp.reshape(nb_ref[...], (b, b))
    nbf = jnp.where(nb != 0, 1.0, 0.0)
    mmat = nbf * (1.0 - dup)
    cnt = jnp.maximum(jnp.sum(mmat, axis=1, keepdims=True), 1.0)
    neigh = jnp.sum(mmat * g_ref[...], axis=1, keepdims=True) / cnt
    loss = jnp.mean(-_ALPHA * lpt_ref[...] - (1.0 - _ALPHA) * neigh)
    out_ref[...] = jnp.reshape(loss, (1, 1))


def _tc_combine(g, lpt, idxr, nb, interpret=False):
    return pl.pallas_call(
        _tc_combine_body,
        out_shape=jax.ShapeDtypeStruct((1, 1), jnp.float32),
        interpret=interpret,
    )(g, lpt, idxr, nb)


def kernel(inputs, inputs_logits, targets, indexes, neighbors, neighbor_dists):
    del inputs, neighbor_dists
    B, _ = inputs_logits.shape
    S = neighbors.shape[1]
    (nb,) = _make_sc(B, S)(neighbors, indexes)
    g, lpt = _tc_dense(inputs_logits, targets.reshape(B, 1))
    loss = _tc_combine(g, lpt, indexes.reshape(1, B), nb)
    return loss[0, 0]
